# baseline (speedup 1.0000x reference)
import math

import jax
import jax.numpy as jnp
from jax.experimental import pallas as pl
from jax.experimental.pallas import tpu as pltpu

_B, _N, _D = 4, 8192, 768
_K = 1024
_ROWS = 512


def _score_body(x_ref, mu_ref, var_ref, g_ref, dt_ref, gamma_ref, beta_ref,
                w_ref, b_ref, s_ref):
    xb = x_ref[...]
    mu = mu_ref[...]
    var = var_ref[...]
    xn = (xb - mu) / jnp.sqrt(var + 1e-5) * gamma_ref[...] + beta_ref[...]
    s128 = jax.lax.dot_general(
        xn.astype(jnp.bfloat16), w_ref[...].astype(jnp.bfloat16),
        (((1,), (1,)), ((), ())), preferred_element_type=jnp.float32)
    s = s128[:, 0:1] + b_ref[...]
    s = s / math.sqrt(2.0)
    s = s + g_ref[...]
    s = s + dt_ref[...]
    s_ref[...] = s


def _scores(x2, mu2, var2, g2, dt2, ln_gamma, ln_beta, wpad, b):
    nblk = (_B * _N) // _ROWS
    return pl.pallas_call(
        _score_body,
        grid=(nblk,),
        in_specs=[
            pl.BlockSpec((_ROWS, _D), lambda i: (i, 0)),
            pl.BlockSpec((_ROWS, 1), lambda i: (i, 0)),
            pl.BlockSpec((_ROWS, 1), lambda i: (i, 0)),
            pl.BlockSpec((_ROWS, 1), lambda i: (i, 0)),
            pl.BlockSpec((_ROWS, 1), lambda i: (i, 0)),
            pl.BlockSpec((1, _D), lambda i: (0, 0)),
            pl.BlockSpec((1, _D), lambda i: (0, 0)),
            pl.BlockSpec((128, _D), lambda i: (0, 0)),
            pl.BlockSpec((1, 1), lambda i: (0, 0)),
        ],
        out_specs=pl.BlockSpec((_ROWS, 1), lambda i: (i, 0)),
        out_shape=jax.ShapeDtypeStruct((_B * _N, 1), jnp.float32),
    )(x2, mu2, var2, g2, dt2, ln_gamma, ln_beta, wpad, b)


def kernel(x, ln_gamma, ln_beta, W, b):
    kg = jax.random.key(42)
    u = jax.random.uniform(kg, (_B, _N, 1), dtype=jnp.float32)
    g = -1.0 * jnp.log(-1.0 * jnp.log(u + 1e-20) + 1e-20)
    kd = jax.random.key(7)
    drop = jax.random.bernoulli(kd, 0.9, (_B, _N, 1)).astype(jnp.float32)
    dt = (1.0 - drop) * -1000000000.0

    mu = jnp.mean(x, axis=-1, keepdims=True)
    var = jnp.var(x, axis=-1, keepdims=True)

    x2 = x.reshape(_B * _N, _D)
    wpad = jnp.zeros((128, _D), jnp.float32).at[0].set(W[0])
    s = _scores(
        x2,
        mu.reshape(_B * _N, 1),
        var.reshape(_B * _N, 1),
        g.reshape(_B * _N, 1),
        dt.reshape(_B * _N, 1),
        ln_gamma.reshape(1, _D),
        ln_beta.reshape(1, _D),
        wpad,
        b.reshape(1, 1),
    )
    scores = s.reshape(_B, _N)
    _, idx = jax.lax.top_k(scores, _K)
    sampled = jnp.take_along_axis(x, idx[..., None], axis=1)
    return sampled


# speedup vs baseline: 1.3635x; 1.3635x over previous
import math

import jax
import jax.numpy as jnp
from jax.experimental import pallas as pl
from jax.experimental.pallas import tpu as pltpu

_B, _N, _D = 4, 8192, 768
_K = 1024
_ROWS = 2048


def _score_body(x_ref, mu_ref, var_ref, g_ref, dt_ref, gamma_ref, beta_ref,
                w_ref, b_ref, s_ref):
    xb = x_ref[...]
    mu = mu_ref[...].reshape(_ROWS, 1)
    var = var_ref[...].reshape(_ROWS, 1)
    xn = (xb - mu) / jnp.sqrt(var + 1e-5) * gamma_ref[...] + beta_ref[...]
    s128 = jax.lax.dot_general(
        xn.astype(jnp.bfloat16), w_ref[...].astype(jnp.bfloat16),
        (((1,), (1,)), ((), ())), preferred_element_type=jnp.float32)
    s = s128[:, 0:1] + b_ref[...]
    s = s / math.sqrt(2.0)
    s = s + g_ref[...].reshape(_ROWS, 1)
    s = s + dt_ref[...].reshape(_ROWS, 1)
    s_ref[...] = s.reshape(_ROWS)


def _scores(x2, mu1, var1, g1, dt1, ln_gamma, ln_beta, wpad, b):
    nblk = (_B * _N) // _ROWS
    return pl.pallas_call(
        _score_body,
        grid=(nblk,),
        in_specs=[
            pl.BlockSpec((_ROWS, _D), lambda i: (i, 0)),
            pl.BlockSpec((_ROWS,), lambda i: (i,)),
            pl.BlockSpec((_ROWS,), lambda i: (i,)),
            pl.BlockSpec((_ROWS,), lambda i: (i,)),
            pl.BlockSpec((_ROWS,), lambda i: (i,)),
            pl.BlockSpec((1, _D), lambda i: (0, 0)),
            pl.BlockSpec((1, _D), lambda i: (0, 0)),
            pl.BlockSpec((128, _D), lambda i: (0, 0)),
            pl.BlockSpec((1, 1), lambda i: (0, 0)),
        ],
        out_specs=pl.BlockSpec((_ROWS,), lambda i: (i,)),
        out_shape=jax.ShapeDtypeStruct((_B * _N,), jnp.float32),
    )(x2, mu1, var1, g1, dt1, ln_gamma, ln_beta, wpad, b)


def kernel(x, ln_gamma, ln_beta, W, b):
    kg = jax.random.key(42)
    u = jax.random.uniform(kg, (_B * _N,), dtype=jnp.float32)
    g = -1.0 * jnp.log(-1.0 * jnp.log(u + 1e-20) + 1e-20)
    kd = jax.random.key(7)
    drop = jax.random.bernoulli(kd, 0.9, (_B * _N,)).astype(jnp.float32)
    dt = (1.0 - drop) * -1000000000.0

    mu = jnp.mean(x, axis=-1)
    var = jnp.var(x, axis=-1)

    x2 = x.reshape(_B * _N, _D)
    wpad = jnp.zeros((128, _D), jnp.float32).at[0].set(W[0])
    s = _scores(
        x2,
        mu.reshape(_B * _N),
        var.reshape(_B * _N),
        g,
        dt,
        ln_gamma.reshape(1, _D),
        ln_beta.reshape(1, _D),
        wpad,
        b.reshape(1, 1),
    )
    scores = s.reshape(_B, _N)
    _, idx = jax.lax.top_k(scores, _K)
    sampled = jnp.take_along_axis(x, idx[..., None], axis=1)
    return sampled


# speedup vs baseline: 1.4202x; 1.0416x over previous
import math

import functools

import jax
import jax.numpy as jnp
from jax import lax
from jax.experimental import pallas as pl
from jax.experimental.pallas import tpu as pltpu
from jax.experimental.pallas import tpu_sc as plsc

_B, _N, _D = 4, 8192, 768
_K = 1024
_ROWS = 2048


def _score_body(x_ref, mu_ref, var_ref, g_ref, dt_ref, gamma_ref, beta_ref,
                w_ref, b_ref, s_ref):
    xb = x_ref[...]
    mu = mu_ref[...].reshape(_ROWS, 1)
    var = var_ref[...].reshape(_ROWS, 1)
    xn = (xb - mu) / jnp.sqrt(var + 1e-5) * gamma_ref[...] + beta_ref[...]
    s128 = jax.lax.dot_general(
        xn.astype(jnp.bfloat16), w_ref[...].astype(jnp.bfloat16),
        (((1,), (1,)), ((), ())), preferred_element_type=jnp.float32)
    s = s128[:, 0:1] + b_ref[...]
    s = s / math.sqrt(2.0)
    s = s + g_ref[...].reshape(_ROWS, 1)
    s = s + dt_ref[...].reshape(_ROWS, 1)
    s_ref[...] = s.reshape(_ROWS)


def _scores(x2, mu1, var1, g1, dt1, ln_gamma, ln_beta, wpad, b):
    nblk = (_B * _N) // _ROWS
    return pl.pallas_call(
        _score_body,
        grid=(nblk,),
        in_specs=[
            pl.BlockSpec((_ROWS, _D), lambda i: (i, 0)),
            pl.BlockSpec((_ROWS,), lambda i: (i,)),
            pl.BlockSpec((_ROWS,), lambda i: (i,)),
            pl.BlockSpec((_ROWS,), lambda i: (i,)),
            pl.BlockSpec((_ROWS,), lambda i: (i,)),
            pl.BlockSpec((1, _D), lambda i: (0, 0)),
            pl.BlockSpec((1, _D), lambda i: (0, 0)),
            pl.BlockSpec((128, _D), lambda i: (0, 0)),
            pl.BlockSpec((1, 1), lambda i: (0, 0)),
        ],
        out_specs=pl.BlockSpec((_ROWS,), lambda i: (i,)),
        out_shape=jax.ShapeDtypeStruct((_B * _N,), jnp.float32),
    )(x2, mu1, var1, g1, dt1, ln_gamma, ln_beta, wpad, b)


_NW = 32
_RPW = (_B * _K) // _NW
_CHUNK = 32


def _make_sc_gather():
    mesh = plsc.VectorSubcoreMesh(core_axis_name="c", subcore_axis_name="s")

    @functools.partial(
        pl.kernel, mesh=mesh,
        out_type=jax.ShapeDtypeStruct((_B * _K, _D), jnp.float32),
        scratch_types=[
            pltpu.VMEM((_RPW,), jnp.int32),
            pltpu.VMEM((2, _CHUNK, _D), jnp.float32),
            pltpu.SemaphoreType.DMA,
            pltpu.SemaphoreType.DMA,
            pltpu.SemaphoreType.DMA,
        ],
    )
    def sc_gather(x_hbm, idx_hbm, out_hbm, idx_v, rows_v, gsem0, gsem1, osem):
        wid = lax.axis_index("s") * 2 + lax.axis_index("c")
        base = wid * _RPW
        pltpu.sync_copy(idx_hbm.at[pl.ds(base, _RPW)], idx_v)
        nchunks = _RPW // _CHUNK
        gsems = [gsem0, gsem1]

        def gather_chunk(i):
            return pltpu.make_async_copy(
                x_hbm.at[idx_v.at[pl.ds(i * _CHUNK, _CHUNK)]],
                rows_v.at[i % 2], gsems[i % 2])

        def out_chunk(i):
            return pltpu.make_async_copy(
                rows_v.at[i % 2],
                out_hbm.at[pl.ds(base + i * _CHUNK, _CHUNK)], osem)

        gather_chunk(0).start()
        for i in range(nchunks):
            gather_chunk(i).wait()
            if i + 1 < nchunks:
                if i >= 1:
                    out_chunk(i - 1).wait()
                gather_chunk(i + 1).start()
            out_chunk(i).start()
        out_chunk(nchunks - 2).wait()
        out_chunk(nchunks - 1).wait()

    return sc_gather


def kernel(x, ln_gamma, ln_beta, W, b):
    kg = jax.random.key(42)
    u = jax.random.uniform(kg, (_B * _N,), dtype=jnp.float32)
    g = -1.0 * jnp.log(-1.0 * jnp.log(u + 1e-20) + 1e-20)
    kd = jax.random.key(7)
    drop = jax.random.bernoulli(kd, 0.9, (_B * _N,)).astype(jnp.float32)
    dt = (1.0 - drop) * -1000000000.0

    mu = jnp.mean(x, axis=-1)
    var = jnp.var(x, axis=-1)

    x2 = x.reshape(_B * _N, _D)
    wpad = jnp.zeros((128, _D), jnp.float32).at[0].set(W[0])
    s = _scores(
        x2,
        mu.reshape(_B * _N),
        var.reshape(_B * _N),
        g,
        dt,
        ln_gamma.reshape(1, _D),
        ln_beta.reshape(1, _D),
        wpad,
        b.reshape(1, 1),
    )
    scores = s.reshape(_B, _N)
    _, idx = jax.lax.top_k(scores, _K)
    flat_idx = (idx + jnp.arange(_B, dtype=idx.dtype)[:, None] * _N).reshape(-1)
    sampled = _make_sc_gather()(x2, flat_idx)
    return sampled.reshape(_B, _K, _D)


# speedup vs baseline: 1.4635x; 1.0305x over previous
import math

import functools

import jax
import jax.numpy as jnp
from jax import lax
from jax.experimental import pallas as pl
from jax.experimental.pallas import tpu as pltpu
from jax.experimental.pallas import tpu_sc as plsc

_B, _N, _D = 4, 8192, 768
_K = 1024
_ROWS = 2048


def _score_body(x_ref, mu_ref, var_ref, gamma_ref, beta_ref, w_ref, s_ref):
    xb = x_ref[...]
    mu = mu_ref[...].reshape(_ROWS, 1)
    var = var_ref[...].reshape(_ROWS, 1)
    xn = (xb - mu) / jnp.sqrt(var + 1e-5) * gamma_ref[...] + beta_ref[...]
    s128 = jax.lax.dot_general(
        xn.astype(jnp.bfloat16), w_ref[...].astype(jnp.bfloat16),
        (((1,), (1,)), ((), ())), preferred_element_type=jnp.float32)
    s_ref[...] = s128[:, 0:1].reshape(_ROWS)


def _scores(x2, mu1, var1, ln_gamma, ln_beta, wpad):
    nblk = (_B * _N) // _ROWS
    return pl.pallas_call(
        _score_body,
        grid=(nblk,),
        in_specs=[
            pl.BlockSpec((_ROWS, _D), lambda i: (i, 0)),
            pl.BlockSpec((_ROWS,), lambda i: (i,)),
            pl.BlockSpec((_ROWS,), lambda i: (i,)),
            pl.BlockSpec((1, _D), lambda i: (0, 0)),
            pl.BlockSpec((1, _D), lambda i: (0, 0)),
            pl.BlockSpec((128, _D), lambda i: (0, 0)),
        ],
        out_specs=pl.BlockSpec((_ROWS,), lambda i: (i,)),
        out_shape=jax.ShapeDtypeStruct((_B * _N,), jnp.float32),
    )(x2, mu1, var1, ln_gamma, ln_beta, wpad)


_NW = 32
_RPW = (_B * _K) // _NW
_CHUNK = 32


def _make_sc_gather():
    mesh = plsc.VectorSubcoreMesh(core_axis_name="c", subcore_axis_name="s")

    @functools.partial(
        pl.kernel, mesh=mesh,
        out_type=jax.ShapeDtypeStruct((_B * _K, _D), jnp.float32),
        scratch_types=[
            pltpu.VMEM((_RPW,), jnp.int32),
            pltpu.VMEM((2, _CHUNK, _D), jnp.float32),
            pltpu.SemaphoreType.DMA,
            pltpu.SemaphoreType.DMA,
            pltpu.SemaphoreType.DMA,
        ],
    )
    def sc_gather(x_hbm, idx_hbm, out_hbm, idx_v, rows_v, gsem0, gsem1, osem):
        wid = lax.axis_index("s") * 2 + lax.axis_index("c")
        base = wid * _RPW
        pltpu.sync_copy(idx_hbm.at[pl.ds(base, _RPW)], idx_v)
        nchunks = _RPW // _CHUNK
        gsems = [gsem0, gsem1]

        def gather_chunk(i):
            return pltpu.make_async_copy(
                x_hbm.at[idx_v.at[pl.ds(i * _CHUNK, _CHUNK)]],
                rows_v.at[i % 2], gsems[i % 2])

        def out_chunk(i):
            return pltpu.make_async_copy(
                rows_v.at[i % 2],
                out_hbm.at[pl.ds(base + i * _CHUNK, _CHUNK)], osem)

        gather_chunk(0).start()
        for i in range(nchunks):
            gather_chunk(i).wait()
            if i + 1 < nchunks:
                if i >= 1:
                    out_chunk(i - 1).wait()
                gather_chunk(i + 1).start()
            out_chunk(i).start()
        out_chunk(nchunks - 2).wait()
        out_chunk(nchunks - 1).wait()

    return sc_gather


def kernel(x, ln_gamma, ln_beta, W, b):
    kg = jax.random.key(42)
    u = jax.random.uniform(kg, (_B * _N,), dtype=jnp.float32)
    g = -1.0 * jnp.log(-1.0 * jnp.log(u + 1e-20) + 1e-20)
    kd = jax.random.key(7)
    drop = jax.random.bernoulli(kd, 0.9, (_B * _N,)).astype(jnp.float32)
    dt = (1.0 - drop) * -1000000000.0

    mu = jnp.mean(x, axis=-1)
    var = jnp.var(x, axis=-1)

    x2 = x.reshape(_B * _N, _D)
    wpad = jnp.zeros((128, _D), jnp.float32).at[0].set(W[0])
    s = _scores(
        x2,
        mu.reshape(_B * _N),
        var.reshape(_B * _N),
        ln_gamma.reshape(1, _D),
        ln_beta.reshape(1, _D),
        wpad,
    )
    s = s + b[0]
    s = s / math.sqrt(2.0)
    s = s + g
    s = s + dt
    scores = s.reshape(_B, _N)
    _, idx = jax.lax.top_k(scores, _K)
    flat_idx = (idx + jnp.arange(_B, dtype=idx.dtype)[:, None] * _N).reshape(-1)
    sampled = _make_sc_gather()(x2, flat_idx)
    return sampled.reshape(_B, _K, _D)


# speedup vs baseline: 1.4774x; 1.0095x over previous
import math

import functools

import jax
import jax.numpy as jnp
from jax import lax
from jax.experimental import pallas as pl
from jax.experimental.pallas import tpu as pltpu
from jax.experimental.pallas import tpu_sc as plsc

_B, _N, _D = 4, 8192, 768
_K = 1024
_ROWS = 2048


def _score_body(x_ref, mu_ref, sd_ref, gamma_ref, beta_ref, w_ref, s_ref):
    xb = x_ref[...]
    mu = mu_ref[...].reshape(_ROWS, 1)
    sd = sd_ref[...].reshape(_ROWS, 1)
    xn = (xb - mu) / sd * gamma_ref[...] + beta_ref[...]
    s128 = jax.lax.dot_general(
        xn.astype(jnp.bfloat16), w_ref[...].astype(jnp.bfloat16),
        (((1,), (1,)), ((), ())), preferred_element_type=jnp.float32)
    s_ref[...] = s128[:, 0:1].reshape(_ROWS)


def _scores(x2, mu1, sd1, ln_gamma, ln_beta, wpad):
    nblk = (_B * _N) // _ROWS
    return pl.pallas_call(
        _score_body,
        grid=(nblk,),
        in_specs=[
            pl.BlockSpec((_ROWS, _D), lambda i: (i, 0)),
            pl.BlockSpec((_ROWS,), lambda i: (i,)),
            pl.BlockSpec((_ROWS,), lambda i: (i,)),
            pl.BlockSpec((1, _D), lambda i: (0, 0)),
            pl.BlockSpec((1, _D), lambda i: (0, 0)),
            pl.BlockSpec((128, _D), lambda i: (0, 0)),
        ],
        out_specs=pl.BlockSpec((_ROWS,), lambda i: (i,)),
        out_shape=jax.ShapeDtypeStruct((_B * _N,), jnp.float32),
    )(x2, mu1, sd1, ln_gamma, ln_beta, wpad)


_NW = 32
_RPW = (_B * _K) // _NW
_CHUNK = 32


def _make_sc_gather():
    mesh = plsc.VectorSubcoreMesh(core_axis_name="c", subcore_axis_name="s")

    @functools.partial(
        pl.kernel, mesh=mesh,
        out_type=jax.ShapeDtypeStruct((_B * _K, _D), jnp.float32),
        scratch_types=[
            pltpu.VMEM((_RPW,), jnp.int32),
            pltpu.VMEM((2, _CHUNK, _D), jnp.float32),
            pltpu.SemaphoreType.DMA,
            pltpu.SemaphoreType.DMA,
            pltpu.SemaphoreType.DMA,
        ],
    )
    def sc_gather(x_hbm, idx_hbm, out_hbm, idx_v, rows_v, gsem0, gsem1, osem):
        wid = lax.axis_index("s") * 2 + lax.axis_index("c")
        base = wid * _RPW
        pltpu.sync_copy(idx_hbm.at[pl.ds(base, _RPW)], idx_v)
        nchunks = _RPW // _CHUNK
        gsems = [gsem0, gsem1]

        def gather_chunk(i):
            return pltpu.make_async_copy(
                x_hbm.at[idx_v.at[pl.ds(i * _CHUNK, _CHUNK)]],
                rows_v.at[i % 2], gsems[i % 2])

        def out_chunk(i):
            return pltpu.make_async_copy(
                rows_v.at[i % 2],
                out_hbm.at[pl.ds(base + i * _CHUNK, _CHUNK)], osem)

        gather_chunk(0).start()
        for i in range(nchunks):
            gather_chunk(i).wait()
            if i + 1 < nchunks:
                if i >= 1:
                    out_chunk(i - 1).wait()
                gather_chunk(i + 1).start()
            out_chunk(i).start()
        out_chunk(nchunks - 2).wait()
        out_chunk(nchunks - 1).wait()

    return sc_gather


def kernel(x, ln_gamma, ln_beta, W, b):
    kg = jax.random.key(42)
    u = jax.random.uniform(kg, (_B * _N,), dtype=jnp.float32)
    g = -1.0 * jnp.log(-1.0 * jnp.log(u + 1e-20) + 1e-20)
    kd = jax.random.key(7)
    drop = jax.random.bernoulli(kd, 0.9, (_B * _N,)).astype(jnp.float32)
    dt = (1.0 - drop) * -1000000000.0

    mu = jnp.mean(x, axis=-1)
    var = jnp.var(x, axis=-1)
    sd = jnp.sqrt(var + 1e-5)

    x2 = x.reshape(_B * _N, _D)
    wpad = jnp.zeros((128, _D), jnp.float32).at[0].set(W[0])
    s = _scores(
        x2,
        mu.reshape(_B * _N),
        sd.reshape(_B * _N),
        ln_gamma.reshape(1, _D),
        ln_beta.reshape(1, _D),
        wpad,
    )
    s = s + b[0]
    s = s / math.sqrt(2.0)
    s = s + g
    s = s + dt
    scores = s.reshape(_B, _N)
    _, idx = jax.lax.top_k(scores, _K)
    flat_idx = (idx + jnp.arange(_B, dtype=idx.dtype)[:, None] * _N).reshape(-1)
    sampled = _make_sc_gather()(x2, flat_idx)
    return sampled.reshape(_B, _K, _D)


# speedup vs baseline: 1.4869x; 1.0064x over previous
import math

import functools

import jax
import jax.numpy as jnp
from jax import lax
from jax.experimental import pallas as pl
from jax.experimental.pallas import tpu as pltpu
from jax.experimental.pallas import tpu_sc as plsc

_B, _N, _D = 4, 8192, 768
_K = 1024
_ROWS = 4096


def _score_body(x_ref, mu_ref, sd_ref, gamma_ref, beta_ref, w_ref, s_ref):
    xb = x_ref[...]
    mu = mu_ref[...].reshape(_ROWS, 1)
    sd = sd_ref[...].reshape(_ROWS, 1)
    xn = (xb - mu) / sd * gamma_ref[...] + beta_ref[...]
    s128 = jax.lax.dot_general(
        xn.astype(jnp.bfloat16), w_ref[...].astype(jnp.bfloat16),
        (((1,), (1,)), ((), ())), preferred_element_type=jnp.float32)
    s_ref[...] = s128[:, 0:1].reshape(_ROWS)


def _scores(x2, mu1, sd1, ln_gamma, ln_beta, wpad):
    nblk = (_B * _N) // _ROWS
    return pl.pallas_call(
        _score_body,
        grid=(nblk,),
        in_specs=[
            pl.BlockSpec((_ROWS, _D), lambda i: (i, 0)),
            pl.BlockSpec((_ROWS,), lambda i: (i,)),
            pl.BlockSpec((_ROWS,), lambda i: (i,)),
            pl.BlockSpec((1, _D), lambda i: (0, 0)),
            pl.BlockSpec((1, _D), lambda i: (0, 0)),
            pl.BlockSpec((128, _D), lambda i: (0, 0)),
        ],
        out_specs=pl.BlockSpec((_ROWS,), lambda i: (i,)),
        out_shape=jax.ShapeDtypeStruct((_B * _N,), jnp.float32),
    )(x2, mu1, sd1, ln_gamma, ln_beta, wpad)


_NW = 32
_RPW = (_B * _K) // _NW
_CHUNK = 32


def _make_sc_gather():
    mesh = plsc.VectorSubcoreMesh(core_axis_name="c", subcore_axis_name="s")

    @functools.partial(
        pl.kernel, mesh=mesh,
        out_type=jax.ShapeDtypeStruct((_B * _K, _D), jnp.float32),
        scratch_types=[
            pltpu.VMEM((_RPW,), jnp.int32),
            pltpu.VMEM((2, _CHUNK, _D), jnp.float32),
            pltpu.SemaphoreType.DMA,
            pltpu.SemaphoreType.DMA,
            pltpu.SemaphoreType.DMA,
        ],
    )
    def sc_gather(x_hbm, idx_hbm, out_hbm, idx_v, rows_v, gsem0, gsem1, osem):
        wid = lax.axis_index("s") * 2 + lax.axis_index("c")
        base = wid * _RPW
        pltpu.sync_copy(idx_hbm.at[pl.ds(base, _RPW)], idx_v)
        nchunks = _RPW // _CHUNK
        gsems = [gsem0, gsem1]

        def gather_chunk(i):
            return pltpu.make_async_copy(
                x_hbm.at[idx_v.at[pl.ds(i * _CHUNK, _CHUNK)]],
                rows_v.at[i % 2], gsems[i % 2])

        def out_chunk(i):
            return pltpu.make_async_copy(
                rows_v.at[i % 2],
                out_hbm.at[pl.ds(base + i * _CHUNK, _CHUNK)], osem)

        gather_chunk(0).start()
        for i in range(nchunks):
            gather_chunk(i).wait()
            if i + 1 < nchunks:
                if i >= 1:
                    out_chunk(i - 1).wait()
                gather_chunk(i + 1).start()
            out_chunk(i).start()
        out_chunk(nchunks - 2).wait()
        out_chunk(nchunks - 1).wait()

    return sc_gather


def kernel(x, ln_gamma, ln_beta, W, b):
    kg = jax.random.key(42)
    u = jax.random.uniform(kg, (_B * _N,), dtype=jnp.float32)
    g = -1.0 * jnp.log(-1.0 * jnp.log(u + 1e-20) + 1e-20)
    kd = jax.random.key(7)
    drop = jax.random.bernoulli(kd, 0.9, (_B * _N,)).astype(jnp.float32)
    dt = (1.0 - drop) * -1000000000.0

    mu = jnp.mean(x, axis=-1)
    var = jnp.var(x, axis=-1)
    sd = jnp.sqrt(var + 1e-5)

    x2 = x.reshape(_B * _N, _D)
    wpad = jnp.zeros((128, _D), jnp.float32).at[0].set(W[0])
    s = _scores(
        x2,
        mu.reshape(_B * _N),
        sd.reshape(_B * _N),
        ln_gamma.reshape(1, _D),
        ln_beta.reshape(1, _D),
        wpad,
    )
    s = s + b[0]
    s = s / math.sqrt(2.0)
    s = s + g
    s = s + dt
    scores = s.reshape(_B, _N)
    _, idx = jax.lax.top_k(scores, _K)
    flat_idx = (idx + jnp.arange(_B, dtype=idx.dtype)[:, None] * _N).reshape(-1)
    sampled = _make_sc_gather()(x2, flat_idx)
    return sampled.reshape(_B, _K, _D)
